# baseline (speedup 1.0000x reference)
import functools

import jax
import jax.numpy as jnp
from jax import lax
from jax.experimental import pallas as pl
from jax.experimental.pallas import tpu as pltpu
from jax.experimental.pallas import tpu_sc as plsc

YEAR_DIM = 128
MONTH_PAD = 16
D_MODEL = 32
HALF = D_MODEL // 2
B_ROWS = 16384
L_SEQ = 200
N_ROWS = B_ROWS * L_SEQ

NW = 32
KB_TOTAL = B_ROWS // 128
KB_PER_W = KB_TOTAL // NW
TILE_ROWS = 128 * L_SEQ
SUB_IDX = 4
SUB_ROWS = SUB_IDX * 128
SUBS_PER_TILE = L_SEQ // SUB_IDX
PAIRS_PER_TILE = SUBS_PER_TILE // 2
N_TILES128 = N_ROWS // 128


def _table_body(yt_ref, mt_ref, w_ref, b_ref, out_ref):
    yt = yt_ref[...]
    mt = mt_ref[...]
    w = w_ref[...]
    b = b_ref[...]
    dn = (((1,), (1,)), ((), ()))
    yp = lax.dot_general(yt, w[:, :HALF], dn,
                         preferred_element_type=jnp.float32,
                         precision=lax.Precision.HIGHEST)
    mp = lax.dot_general(mt, w[:, HALF:], dn,
                         preferred_element_type=jnp.float32,
                         precision=lax.Precision.HIGHEST) + b
    comb = yp[:, None, :] + mp[None, :, :]
    out_ref[...] = comb.reshape(YEAR_DIM * MONTH_PAD, D_MODEL)


def _build_table(year_table, month_table_padded, w, b2d):
    return pl.pallas_call(
        _table_body,
        out_shape=jax.ShapeDtypeStruct((YEAR_DIM * MONTH_PAD, D_MODEL),
                                       jnp.float32),
    )(year_table, month_table_padded, w, b2d)


def _fuse_body(yi_ref, mi_ref, out_ref):
    ci = yi_ref[...] * MONTH_PAD + mi_ref[...]
    out_ref[...] = ci.reshape(1, L_SEQ, 128)


def _fuse_indices(yi_t, mi_t):
    return pl.pallas_call(
        _fuse_body,
        grid=(KB_TOTAL,),
        in_specs=[
            pl.BlockSpec((L_SEQ, 128), lambda k: (0, k)),
            pl.BlockSpec((L_SEQ, 128), lambda k: (0, k)),
        ],
        out_specs=pl.BlockSpec((1, L_SEQ, 128), lambda k: (k, 0, 0)),
        out_shape=jax.ShapeDtypeStruct((KB_TOTAL, L_SEQ, 128), jnp.int32),
    )(yi_t, mi_t)


_SC_MESH = plsc.VectorSubcoreMesh(core_axis_name="c", subcore_axis_name="s")


def _iota16():
    return lax.broadcasted_iota(jnp.int32, (16,), 0)


@functools.partial(
    pl.kernel,
    out_type=jax.ShapeDtypeStruct((4, N_TILES128, 8, 128), jnp.float32),
    mesh=_SC_MESH,
    compiler_params=pltpu.CompilerParams(use_tc_tiling_on_sc=False,
                                         needs_layout_passes=False),
    scratch_types=[
        pltpu.VMEM((L_SEQ, 128), jnp.int32),
        pltpu.VMEM((L_SEQ, 128), jnp.int32),
        pltpu.VMEM((SUB_ROWS, D_MODEL), jnp.float32),
        pltpu.VMEM((SUB_ROWS, D_MODEL), jnp.float32),
        pltpu.VMEM((4, SUB_IDX, 8, 128), jnp.float32),
        pltpu.VMEM((4, SUB_IDX, 8, 128), jnp.float32),
        pltpu.SemaphoreType.DMA,
        pltpu.SemaphoreType.DMA,
        pltpu.SemaphoreType.DMA,
        pltpu.SemaphoreType.DMA,
    ],
)
def _sc_lookup(table_hbm, ci3_hbm, out_hbm, tile_v, cir_v, rows0, rows1,
               tb0, tb1, sem_g0, sem_g1, sem_w0, sem_w1):
    wid = lax.axis_index("s") * 2 + lax.axis_index("c")
    iota = _iota16()
    rows_b = (rows0, rows1)
    tb_b = (tb0, tb1)
    sem_g = (sem_g0, sem_g1)
    sem_w = (sem_w0, sem_w1)

    def issue_gathers(sub, h):
        for j in range(SUB_IDX):
            pltpu.async_copy(
                table_hbm.at[cir_v.at[sub * SUB_IDX + j]],
                rows_b[h].at[pl.ds(j * 128, 128)],
                sem_g[h],
            )

    def wait_gathers(h):
        for j in range(SUB_IDX):
            pltpu.make_async_copy(
                table_hbm.at[cir_v.at[j]],
                rows_b[h].at[pl.ds(j * 128, 128)],
                sem_g[h],
            ).wait()

    def transpose_sub(h):
        rv = rows_b[h]
        tv = tb_b[h]

        def band_body(band, c0):
            for cc in range(8):
                cols_i = jnp.zeros((16,), jnp.int32) + (band * 8 + cc)
                for tt in range(SUB_IDX):
                    for g in range(8):
                        rows_i = iota + (tt * 128 + g * 16)
                        vals = plsc.load_gather(rv, [rows_i, cols_i])
                        tv[band, tt, cc, pl.ds(g * 16, 16)] = vals
            return c0

        lax.fori_loop(0, 4, band_body, 0)

    def wb_dst(t0):
        return out_hbm.at[:, pl.ds(t0, SUB_IDX)]

    def tile_body(t, carry):
        kb = wid * KB_PER_W + t
        pltpu.sync_copy(ci3_hbm.at[kb], tile_v)

        def pair_body(p, c2):
            b0 = p * 2
            for k in range(25):
                if k < 12:
                    rows_i = iota + (16 * k)
                    cols_i = jnp.zeros((16,), jnp.int32) + b0
                elif k == 12:
                    la = iota + 192
                    wrap = la >= L_SEQ
                    rows_i = la - jnp.where(wrap, L_SEQ, 0)
                    cols_i = jnp.where(wrap, 1, 0) + b0
                else:
                    rows_i = iota + (16 * k - L_SEQ)
                    cols_i = jnp.zeros((16,), jnp.int32) + (b0 + 1)
                vals = plsc.load_gather(tile_v, [rows_i, cols_i])
                r0 = p * 400 + 16 * k
                cir_v[r0 // 128, pl.ds(lax.rem(r0, 128), 16)] = vals
            return c2

        lax.fori_loop(0, 64, pair_body, 0)

        tile_t0 = kb * (TILE_ROWS // 128)
        issue_gathers(0, 0)

        def pipe_body(pg, c3):
            for h in (0, 1):
                sub = pg * 2 + h
                nxt = sub + 1

                @pl.when(nxt < SUBS_PER_TILE)
                def _prefetch():
                    issue_gathers(nxt, 1 - h)

                wait_gathers(h)

                @pl.when(jnp.logical_or(t > 0, sub > 1))
                def _wait_wb():
                    pltpu.make_async_copy(tb_b[h], wb_dst(0), sem_w[h]).wait()

                transpose_sub(h)
                pltpu.async_copy(
                    tb_b[h], wb_dst(tile_t0 + sub * SUB_IDX), sem_w[h])
            return c3

        lax.fori_loop(0, PAIRS_PER_TILE, pipe_body, 0)
        return carry

    lax.fori_loop(0, KB_PER_W, tile_body, 0)
    pltpu.make_async_copy(tb0, wb_dst(0), sem_w0).wait()
    pltpu.make_async_copy(tb1, wb_dst(0), sem_w1).wait()


def kernel(year_indices, month_indices, year_table, month_table, W, b):
    mt_pad = jnp.zeros((MONTH_PAD, HALF), jnp.float32).at[:12].set(month_table)
    table = _build_table(year_table, mt_pad, W, b.reshape(1, D_MODEL))
    ci3 = _fuse_indices(year_indices.T.astype(jnp.int32),
                        month_indices.T.astype(jnp.int32))
    out4 = _sc_lookup(table, ci3)
    return out4.transpose(1, 3, 0, 2).reshape(N_ROWS, D_MODEL)


# speedup vs baseline: 1.2885x; 1.2885x over previous
import functools

import jax
import jax.numpy as jnp
from jax import lax
from jax.experimental import pallas as pl
from jax.experimental.pallas import tpu as pltpu
from jax.experimental.pallas import tpu_sc as plsc

YEAR_DIM = 128
MONTH_PAD = 16
D_MODEL = 32
HALF = D_MODEL // 2
B_ROWS = 16384
L_SEQ = 200
N_ROWS = B_ROWS * L_SEQ

NW = 32
KB_TOTAL = B_ROWS // 128
KB_PER_W = KB_TOTAL // NW
TILE_ROWS = 128 * L_SEQ
SUB_IDX = 2
SUB_ROWS = SUB_IDX * 128
SUBS_PER_TILE = L_SEQ // SUB_IDX
PAIRS_PER_TILE = SUBS_PER_TILE // 2
N_TILES128 = N_ROWS // 128


def _table_body(yt_ref, mt_ref, w_ref, b_ref, out_ref):
    yt = yt_ref[...]
    mt = mt_ref[...]
    w = w_ref[...]
    b = b_ref[...]
    dn = (((1,), (1,)), ((), ()))
    yp = lax.dot_general(yt, w[:, :HALF], dn,
                         preferred_element_type=jnp.float32,
                         precision=lax.Precision.HIGHEST)
    mp = lax.dot_general(mt, w[:, HALF:], dn,
                         preferred_element_type=jnp.float32,
                         precision=lax.Precision.HIGHEST) + b
    comb = yp[:, None, :] + mp[None, :, :]
    out_ref[...] = comb.reshape(YEAR_DIM * MONTH_PAD, D_MODEL)


def _build_table(year_table, month_table_padded, w, b2d):
    return pl.pallas_call(
        _table_body,
        out_shape=jax.ShapeDtypeStruct((YEAR_DIM * MONTH_PAD, D_MODEL),
                                       jnp.float32),
    )(year_table, month_table_padded, w, b2d)


def _fuse_body(yi_ref, mi_ref, out_ref):
    ci = yi_ref[...] * MONTH_PAD + mi_ref[...]
    out_ref[...] = ci.reshape(1, L_SEQ, 128)


def _fuse_indices(yi_t, mi_t):
    return pl.pallas_call(
        _fuse_body,
        grid=(KB_TOTAL,),
        in_specs=[
            pl.BlockSpec((L_SEQ, 128), lambda k: (0, k)),
            pl.BlockSpec((L_SEQ, 128), lambda k: (0, k)),
        ],
        out_specs=pl.BlockSpec((1, L_SEQ, 128), lambda k: (k, 0, 0)),
        out_shape=jax.ShapeDtypeStruct((KB_TOTAL, L_SEQ, 128), jnp.int32),
    )(yi_t, mi_t)


_SC_MESH = plsc.VectorSubcoreMesh(core_axis_name="c", subcore_axis_name="s")


def _iota16():
    return lax.broadcasted_iota(jnp.int32, (16,), 0)


@functools.partial(
    pl.kernel,
    out_type=jax.ShapeDtypeStruct((4, N_TILES128, 8, 128), jnp.float32),
    mesh=_SC_MESH,
    compiler_params=pltpu.CompilerParams(use_tc_tiling_on_sc=False,
                                         needs_layout_passes=False),
    scratch_types=[
        pltpu.VMEM((L_SEQ, 128), jnp.int32),
        pltpu.VMEM((L_SEQ, 128), jnp.int32),
        pltpu.VMEM((SUB_ROWS, D_MODEL), jnp.float32),
        pltpu.VMEM((SUB_ROWS, D_MODEL), jnp.float32),
        pltpu.VMEM((4, SUB_IDX, 8, 128), jnp.float32),
        pltpu.VMEM((4, SUB_IDX, 8, 128), jnp.float32),
        pltpu.SemaphoreType.DMA,
        pltpu.SemaphoreType.DMA,
        pltpu.SemaphoreType.DMA,
        pltpu.SemaphoreType.DMA,
    ],
)
def _sc_lookup(table_hbm, ci3_hbm, out_hbm, tile_v, cir_v, rows0, rows1,
               tb0, tb1, sem_g0, sem_g1, sem_w0, sem_w1):
    wid = lax.axis_index("s") * 2 + lax.axis_index("c")
    iota = _iota16()
    rows_b = (rows0, rows1)
    tb_b = (tb0, tb1)
    sem_g = (sem_g0, sem_g1)
    sem_w = (sem_w0, sem_w1)

    def issue_gathers(sub, h):
        for j in range(SUB_IDX):
            pltpu.async_copy(
                table_hbm.at[cir_v.at[sub * SUB_IDX + j]],
                rows_b[h].at[pl.ds(j * 128, 128)],
                sem_g[h],
            )

    def wait_gathers(h):
        for j in range(SUB_IDX):
            pltpu.make_async_copy(
                table_hbm.at[cir_v.at[j]],
                rows_b[h].at[pl.ds(j * 128, 128)],
                sem_g[h],
            ).wait()

    def transpose_sub(h):
        rv = rows_b[h]
        tv = tb_b[h]
        row_vecs = [iota + (tt * 128 + g * 16)
                    for tt in range(SUB_IDX) for g in range(8)]
        for band in range(4):
            for cc in range(8):
                cols_i = jnp.zeros((16,), jnp.int32) + (band * 8 + cc)
                for tt in range(SUB_IDX):
                    vals = [plsc.load_gather(rv, [row_vecs[tt * 8 + g], cols_i])
                            for g in range(8)]
                    for g in range(8):
                        tv[band, tt, cc, pl.ds(g * 16, 16)] = vals[g]

    def wb_dst(t0):
        return out_hbm.at[:, pl.ds(t0, SUB_IDX)]

    def tile_body(t, carry):
        kb = wid * KB_PER_W + t
        pltpu.sync_copy(ci3_hbm.at[kb], tile_v)

        def row_body(R, c2):
            r_lo = R * 128
            addrs = [iota + (r_lo + 16 * m) for m in range(8)]
            vals = [
                plsc.load_gather(tile_v, [lax.rem(a, L_SEQ), a // L_SEQ])
                for a in addrs
            ]
            for m in range(8):
                cir_v[R, pl.ds(16 * m, 16)] = vals[m]
            return c2

        lax.fori_loop(0, L_SEQ, row_body, 0)

        tile_t0 = kb * (TILE_ROWS // 128)
        issue_gathers(0, 0)

        def pipe_body(pg, c3):
            for h in (0, 1):
                sub = pg * 2 + h
                nxt = sub + 1

                @pl.when(nxt < SUBS_PER_TILE)
                def _prefetch():
                    issue_gathers(nxt, 1 - h)

                wait_gathers(h)

                @pl.when(jnp.logical_or(t > 0, sub > 1))
                def _wait_wb():
                    pltpu.make_async_copy(tb_b[h], wb_dst(0), sem_w[h]).wait()

                transpose_sub(h)
                pltpu.async_copy(
                    tb_b[h], wb_dst(tile_t0 + sub * SUB_IDX), sem_w[h])
            return c3

        lax.fori_loop(0, PAIRS_PER_TILE, pipe_body, 0)
        return carry

    lax.fori_loop(0, KB_PER_W, tile_body, 0)
    pltpu.make_async_copy(tb0, wb_dst(0), sem_w0).wait()
    pltpu.make_async_copy(tb1, wb_dst(0), sem_w1).wait()


def kernel(year_indices, month_indices, year_table, month_table, W, b):
    mt_pad = jnp.zeros((MONTH_PAD, HALF), jnp.float32).at[:12].set(month_table)
    table = _build_table(year_table, mt_pad, W, b.reshape(1, D_MODEL))
    ci3 = _fuse_indices(year_indices.T.astype(jnp.int32),
                        month_indices.T.astype(jnp.int32))
    out4 = _sc_lookup(table, ci3)
    return out4.transpose(1, 3, 0, 2).reshape(N_ROWS, D_MODEL)
